# baseline (speedup 1.0000x reference)
import dataclasses
import functools
import math

import jax
import jax.numpy as jnp
from jax import lax
from jax.experimental import pallas as pl
from jax.experimental.pallas import tpu as pltpu
from jax.experimental.pallas import tpu_sc as plsc

NC = 2
NS = 16
L = 16
FBLK = 64
SBLK = 1024
SEL = 160


def _ea_tc_kernel(attr_ref, w_ref, b_ref, o0_ref, o1_ref):
    attr = attr_ref[...]
    o0_ref[...] = (
        jnp.dot(attr, w_ref[0], preferred_element_type=jnp.float32,
                precision=lax.Precision.HIGHEST)
        + b_ref[0, 0]
    )
    o1_ref[...] = (
        jnp.dot(attr, w_ref[1], preferred_element_type=jnp.float32,
                precision=lax.Precision.HIGHEST)
        + b_ref[1, 0]
    )


def _sc_kernel(e_pad, npt, n_out,
               x_hbm, gidx_hbm, sidx_hbm, ea_hbm, out_hbm,
               gscan0, sscan0, gscan1, sscan1, gsel_v, ssel_v, psel_v,
               gfire0, sfire0, pfire0, rows0, ea0,
               gfire1, sfire1, pfire1, rows1, ea1,
               cnt_s, sem_s0, sem_s1, sem_f0, sem_f1, acc):
    s = lax.axis_index("subcore")
    d = x_hbm.shape[1]
    lo = s * npt
    junk = n_out + s
    lanes = jnp.arange(L, dtype=jnp.int32)
    nblk = e_pad // SBLK
    bufs = ((gfire0, sfire0, pfire0, rows0, ea0, sem_f0),
            (gfire1, sfire1, pfire1, rows1, ea1, sem_f1))

    nzb = rows0.shape[0]

    @pl.loop(0, nzb)
    def _(i):
        for j in range(d // L):
            rows0[i, pl.ds(j * L, L)] = jnp.zeros((L,), jnp.float32)

    for off in sorted({min(j * nzb, npt - nzb) for j in range(-(-npt // nzb))}):
        pltpu.sync_copy(rows0, acc.at[pl.ds(lo + off, nzb)])

    cnt_s[0] = 0
    cnt_s[1] = 0
    cnt_s[2] = 0
    cnt_s[3] = 0

    def relu_rows(rows_v, ea_v):
        @pl.loop(0, FBLK)
        def _(e):
            for j in range(d // L):
                sl = pl.ds(j * L, L)
                rows_v[e, sl] = jnp.maximum(rows_v[e, sl] + ea_v[e, sl], 0.0)

    def flush(p):
        gfire_v, sfire_v, pfire_v, rows_v, ea_v, sem = bufs[p]

        @pl.when(cnt_s[1 + p] == 1)
        def _():
            pltpu.make_async_copy(x_hbm.at[gfire_v.at[0]], rows_v, sem).wait()
            pltpu.make_async_copy(ea_hbm.at[pfire_v.at[0]], ea_v, sem).wait()
            relu_rows(rows_v, ea_v)
            pltpu.sync_copy(rows_v, acc.at[sfire_v.at[0]], add=True)
            cnt_s[1 + p] = 0

    def fire(p):
        gfire_v, sfire_v, pfire_v, rows_v, ea_v, sem = bufs[p]
        for j in range(FBLK // L):
            sl = pl.ds(j * L, L)
            gfire_v[0, sl] = gsel_v[sl]
            sfire_v[0, sl] = ssel_v[sl]
            pfire_v[0, sl] = psel_v[sl]
        pltpu.async_copy(x_hbm.at[gfire_v.at[0]], rows_v, sem)
        pltpu.async_copy(ea_hbm.at[pfire_v.at[0]], ea_v, sem)
        for j in range((SEL - FBLK) // L):
            sl = pl.ds(j * L, L)
            gsel_v[sl] = gsel_v[pl.ds(FBLK + j * L, L)]
            ssel_v[sl] = ssel_v[pl.ds(FBLK + j * L, L)]
            psel_v[sl] = psel_v[pl.ds(FBLK + j * L, L)]
        cnt_s[0] = cnt_s[0] - FBLK
        cnt_s[1 + p] = 1

    def maybe_fire():
        @pl.when(cnt_s[0] >= FBLK)
        def _():
            par = cnt_s[3]

            @pl.when(par == 0)
            def _():
                flush(0)
                fire(0)
                cnt_s[3] = 1

            @pl.when(par == 1)
            def _():
                flush(1)
                fire(1)
                cnt_s[3] = 0

    def scan_block(blk, gscan_v, sscan_v):
        @pl.loop(0, SBLK // 128)
        def _(sb):
            for half in range(2):
                for k in range(4):
                    off = sb * 128 + half * 64 + k * L
                    dstv = sscan_v[pl.ds(off, L)]
                    srcv = gscan_v[pl.ds(off, L)]
                    m = (dstv >= lo) & (dstv < lo + npt)
                    ptr = cnt_s[0]
                    plsc.store_compressed(gsel_v.at[pl.ds(ptr, L)], srcv,
                                          mask=m)
                    plsc.store_compressed(ssel_v.at[pl.ds(ptr, L)], dstv,
                                          mask=m)
                    posv = (blk * SBLK + off) + lanes
                    plsc.store_compressed(psel_v.at[pl.ds(ptr, L)], posv,
                                          mask=m)
                    cnt_s[0] = ptr + plsc.all_reduce_population_count(m)[0]
                maybe_fire()

    def scan_issue(blk, gscan_v, sscan_v, sem):
        pltpu.async_copy(
            gidx_hbm.at[pl.ds(blk * SBLK, SBLK)], gscan_v, sem)
        pltpu.async_copy(
            sidx_hbm.at[pl.ds(blk * SBLK, SBLK)], sscan_v, sem)

    def scan_wait(blk, gscan_v, sscan_v, sem):
        pltpu.make_async_copy(
            gidx_hbm.at[pl.ds(blk * SBLK, SBLK)], gscan_v, sem).wait()
        pltpu.make_async_copy(
            sidx_hbm.at[pl.ds(blk * SBLK, SBLK)], sscan_v, sem).wait()

    scan_issue(0, gscan0, sscan0, sem_s0)

    @pl.loop(0, nblk // 2)
    def _(i):
        blk0 = 2 * i
        scan_wait(blk0, gscan0, sscan0, sem_s0)
        scan_issue(blk0 + 1, gscan1, sscan1, sem_s1)
        scan_block(blk0, gscan0, sscan0)
        scan_wait(blk0 + 1, gscan1, sscan1, sem_s1)

        @pl.when(blk0 + 2 < nblk)
        def _():
            scan_issue(blk0 + 2, gscan0, sscan0, sem_s0)

        scan_block(blk0 + 1, gscan1, sscan1)

    flush(0)
    flush(1)
    cnt = cnt_s[0]
    for j in range(FBLK // L):
        sl = pl.ds(j * L, L)
        live = (lanes + (j * L)) < cnt
        gfire0[0, sl] = jnp.where(live, gsel_v[sl], 0)
        sfire0[0, sl] = jnp.where(live, ssel_v[sl], junk)
        pfire0[0, sl] = jnp.where(live, psel_v[sl], 0)
    pltpu.sync_copy(x_hbm.at[gfire0.at[0]], rows0)
    pltpu.sync_copy(ea_hbm.at[pfire0.at[0]], ea0)
    relu_rows(rows0, ea0)
    pltpu.sync_copy(rows0, acc.at[sfire0.at[0]], add=True)

    pltpu.sync_copy(acc.at[pl.ds(lo, npt)], out_hbm.at[pl.ds(lo, npt)])


def _gelu(h):
    return 0.5 * h * (1.0 + lax.erf(h * (1.0 / math.sqrt(2.0))))


def _post_tc_kernel(x_ref, a0_ref, a1_ref, w_ref, b_ref, o_ref):
    xb = x_ref[...]
    hp = lax.Precision.HIGHEST
    hin = xb + a0_ref[...]
    hin = jnp.dot(hin, w_ref[0], preferred_element_type=jnp.float32,
                  precision=hp) + b_ref[0]
    hin = _gelu(hin)
    hin = jnp.dot(hin, w_ref[1], preferred_element_type=jnp.float32,
                  precision=hp) + b_ref[1]
    hout = xb + a1_ref[...]
    hout = jnp.dot(hout, w_ref[2], preferred_element_type=jnp.float32,
                   precision=hp) + b_ref[2]
    hout = _gelu(hout)
    hout = jnp.dot(hout, w_ref[3], preferred_element_type=jnp.float32,
                   precision=hp) + b_ref[3]
    o_ref[...] = (0.5 * (hin + hout)
                  + jnp.dot(xb, w_ref[4], preferred_element_type=jnp.float32,
                            precision=hp)
                  + b_ref[4])


def kernel(x, edge_index, edge_attr, We_in, be_in, W1_in, b1_in, W2_in, b2_in,
           We_out, be_out, W1_out, b1_out, W2_out, b2_out, Wr, br):
    n, d = x.shape
    e = edge_index.shape[1]
    de = edge_attr.shape[1]

    blk_e = 2048
    e_pad = -(-e // blk_e) * blk_e
    pad = e_pad - e
    npt = -(-n // (NS * 8)) * 8
    n_out = npt * NS

    src = edge_index[0].astype(jnp.int32)
    dst = edge_index[1].astype(jnp.int32)
    zpad = jnp.zeros((pad,), jnp.int32)
    jpad = jnp.full((pad,), n, jnp.int32)
    gidx0 = jnp.concatenate([src, zpad])
    sidx0 = jnp.concatenate([dst, jpad])
    gidx1 = jnp.concatenate([dst, zpad])
    sidx1 = jnp.concatenate([src, jpad])

    attr_pad = jnp.concatenate(
        [edge_attr, jnp.zeros((pad, de), jnp.float32)], axis=0)
    We_all = jnp.stack([We_in, We_out])
    be_all = jnp.stack([be_in, be_out]).reshape(2, 1, d)

    ea0, ea1 = pl.pallas_call(
        _ea_tc_kernel,
        grid=(e_pad // blk_e,),
        in_specs=[
            pl.BlockSpec((blk_e, de), lambda i: (i, 0)),
            pl.BlockSpec((2, de, d), lambda i: (0, 0, 0)),
            pl.BlockSpec((2, 1, d), lambda i: (0, 0, 0)),
        ],
        out_specs=[pl.BlockSpec((blk_e, d), lambda i: (i, 0)),
                   pl.BlockSpec((blk_e, d), lambda i: (i, 0))],
        out_shape=[jax.ShapeDtypeStruct((e_pad, d), jnp.float32),
                   jax.ShapeDtypeStruct((e_pad, d), jnp.float32)],
    )(attr_pad, We_all, be_all)

    sc_params = pltpu.CompilerParams()
    if "needs_layout_passes" in pltpu.CompilerParams.__dataclass_fields__:
        sc_params = dataclasses.replace(sc_params, needs_layout_passes=False)

    def sc_call(xv, gidx, sidx, ea):
        mesh = plsc.VectorSubcoreMesh(core_axis_name="core",
                                      subcore_axis_name="subcore",
                                      num_cores=1, num_subcores=NS)
        call = pl.kernel(
            functools.partial(_sc_kernel, e_pad, npt, n_out),
            out_type=jax.ShapeDtypeStruct((n_out, d), jnp.float32),
            mesh=mesh,
            scratch_types=(
                [pltpu.VMEM((SBLK,), jnp.int32)] * 4
                + [pltpu.VMEM((SEL,), jnp.int32)] * 3
                + [pltpu.VMEM((1, FBLK), jnp.int32)] * 3
                + [pltpu.VMEM((FBLK, d), jnp.float32)] * 2
                + [pltpu.VMEM((1, FBLK), jnp.int32)] * 3
                + [pltpu.VMEM((FBLK, d), jnp.float32)] * 2
                + [pltpu.SMEM((4,), jnp.int32)]
                + [pltpu.SemaphoreType.DMA] * 4
                + [pltpu.VMEM_SHARED((n_out + NS, d), jnp.float32)]
            ),
            compiler_params=sc_params,
        )
        return call(xv, gidx, sidx, ea)

    aggr0 = sc_call(x, gidx0, sidx0, ea0)
    aggr1 = sc_call(x, gidx1, sidx1, ea1)

    W_all = jnp.stack([W1_in, W2_in, W1_out, W2_out, Wr])
    b_all = jnp.stack([b1_in, b2_in, b1_out, b2_out, br])

    blk_n = 2000
    out = pl.pallas_call(
        _post_tc_kernel,
        grid=(n // blk_n,),
        in_specs=[
            pl.BlockSpec((blk_n, d), lambda i: (i, 0)),
            pl.BlockSpec((blk_n, d), lambda i: (i, 0)),
            pl.BlockSpec((blk_n, d), lambda i: (i, 0)),
            pl.BlockSpec((5, d, d), lambda i: (0, 0, 0)),
            pl.BlockSpec((5, d), lambda i: (0, 0)),
        ],
        out_specs=pl.BlockSpec((blk_n, d), lambda i: (i, 0)),
        out_shape=jax.ShapeDtypeStruct((n, d), jnp.float32),
    )(x, aggr0, aggr1, W_all, b_all)
    return out


# speedup vs baseline: 1.3020x; 1.3020x over previous
import dataclasses
import functools
import math

import jax
import jax.numpy as jnp
from jax import lax
from jax.experimental import pallas as pl
from jax.experimental.pallas import tpu as pltpu
from jax.experimental.pallas import tpu_sc as plsc

NC = 2
NS = 16
L = 16
FBLK = 64
SBLK = 1024
SEL = 160


def _ea_tc_kernel(attr_ref, w_ref, b_ref, o_ref):
    o_ref[0] = (
        jnp.dot(attr_ref[...], w_ref[0], preferred_element_type=jnp.float32,
                precision=lax.Precision.HIGHEST)
        + b_ref[0, 0]
    )


def _sc_kernel(e_pad, npt, n_out,
               x_hbm, gidx_hbm, sidx_hbm, ea_hbm, out_hbm,
               gscan0, sscan0, gscan1, sscan1, gsel_v, ssel_v, psel_v,
               gfire0, sfire0, pfire0, rows0, ea0,
               gfire1, sfire1, pfire1, rows1, ea1,
               cnt_s, sem_s0, sem_s1, sem_f0, sem_f1, acc):
    c = lax.axis_index("core")
    s = lax.axis_index("subcore")
    d = x_hbm.shape[1]
    lo = s * npt
    junk = n_out + s
    lanes = jnp.arange(L, dtype=jnp.int32)
    nblk = e_pad // SBLK
    bufs = ((gfire0, sfire0, pfire0, rows0, ea0, sem_f0),
            (gfire1, sfire1, pfire1, rows1, ea1, sem_f1))

    nzb = rows0.shape[0]

    @pl.loop(0, nzb)
    def _(i):
        for j in range(d // L):
            rows0[i, pl.ds(j * L, L)] = jnp.zeros((L,), jnp.float32)

    for off in sorted({min(j * nzb, npt - nzb) for j in range(-(-npt // nzb))}):
        pltpu.sync_copy(rows0, acc.at[pl.ds(lo + off, nzb)])

    cnt_s[0] = 0
    cnt_s[1] = 0
    cnt_s[2] = 0
    cnt_s[3] = 0

    def relu_rows(rows_v, ea_v):
        @pl.loop(0, FBLK)
        def _(e):
            for j in range(d // L):
                sl = pl.ds(j * L, L)
                rows_v[e, sl] = jnp.maximum(rows_v[e, sl] + ea_v[e, sl], 0.0)

    def flush(p):
        gfire_v, sfire_v, pfire_v, rows_v, ea_v, sem = bufs[p]

        @pl.when(cnt_s[1 + p] == 1)
        def _():
            pltpu.make_async_copy(x_hbm.at[gfire_v.at[0]], rows_v, sem).wait()
            pltpu.make_async_copy(ea_hbm.at[pfire_v.at[0]], ea_v, sem).wait()
            relu_rows(rows_v, ea_v)
            pltpu.sync_copy(rows_v, acc.at[sfire_v.at[0]], add=True)
            cnt_s[1 + p] = 0

    def fire(p):
        gfire_v, sfire_v, pfire_v, rows_v, ea_v, sem = bufs[p]
        for j in range(FBLK // L):
            sl = pl.ds(j * L, L)
            gfire_v[0, sl] = gsel_v[sl]
            sfire_v[0, sl] = ssel_v[sl]
            pfire_v[0, sl] = psel_v[sl]
        pltpu.async_copy(x_hbm.at[gfire_v.at[0]], rows_v, sem)
        pltpu.async_copy(ea_hbm.at[pfire_v.at[0]], ea_v, sem)
        for j in range((SEL - FBLK) // L):
            sl = pl.ds(j * L, L)
            gsel_v[sl] = gsel_v[pl.ds(FBLK + j * L, L)]
            ssel_v[sl] = ssel_v[pl.ds(FBLK + j * L, L)]
            psel_v[sl] = psel_v[pl.ds(FBLK + j * L, L)]
        cnt_s[0] = cnt_s[0] - FBLK
        cnt_s[1 + p] = 1

    def maybe_fire():
        @pl.when(cnt_s[0] >= FBLK)
        def _():
            par = cnt_s[3]

            @pl.when(par == 0)
            def _():
                flush(0)
                fire(0)
                cnt_s[3] = 1

            @pl.when(par == 1)
            def _():
                flush(1)
                fire(1)
                cnt_s[3] = 0

    def scan_block(blk, gscan_v, sscan_v):
        @pl.loop(0, SBLK // 128)
        def _(sb):
            for half in range(2):
                for k in range(4):
                    off = sb * 128 + half * 64 + k * L
                    dstv = sscan_v[pl.ds(off, L)]
                    srcv = gscan_v[pl.ds(off, L)]
                    m = (dstv >= lo) & (dstv < lo + npt)
                    ptr = cnt_s[0]
                    plsc.store_compressed(gsel_v.at[pl.ds(ptr, L)], srcv,
                                          mask=m)
                    plsc.store_compressed(ssel_v.at[pl.ds(ptr, L)], dstv,
                                          mask=m)
                    posv = (c * e_pad + blk * SBLK + off) + lanes
                    plsc.store_compressed(psel_v.at[pl.ds(ptr, L)], posv,
                                          mask=m)
                    cnt_s[0] = ptr + plsc.all_reduce_population_count(m)[0]
                maybe_fire()

    def scan_issue(blk, gscan_v, sscan_v, sem):
        pltpu.async_copy(
            gidx_hbm.at[pl.ds(c * e_pad + blk * SBLK, SBLK)], gscan_v, sem)
        pltpu.async_copy(
            sidx_hbm.at[pl.ds(c * e_pad + blk * SBLK, SBLK)], sscan_v, sem)

    def scan_wait(blk, gscan_v, sscan_v, sem):
        pltpu.make_async_copy(
            gidx_hbm.at[pl.ds(c * e_pad + blk * SBLK, SBLK)], gscan_v,
            sem).wait()
        pltpu.make_async_copy(
            sidx_hbm.at[pl.ds(c * e_pad + blk * SBLK, SBLK)], sscan_v,
            sem).wait()

    scan_issue(0, gscan0, sscan0, sem_s0)

    @pl.loop(0, nblk // 2)
    def _(i):
        blk0 = 2 * i
        scan_wait(blk0, gscan0, sscan0, sem_s0)
        scan_issue(blk0 + 1, gscan1, sscan1, sem_s1)
        scan_block(blk0, gscan0, sscan0)
        scan_wait(blk0 + 1, gscan1, sscan1, sem_s1)

        @pl.when(blk0 + 2 < nblk)
        def _():
            scan_issue(blk0 + 2, gscan0, sscan0, sem_s0)

        scan_block(blk0 + 1, gscan1, sscan1)

    flush(0)
    flush(1)
    cnt = cnt_s[0]
    for j in range(FBLK // L):
        sl = pl.ds(j * L, L)
        live = (lanes + (j * L)) < cnt
        gfire0[0, sl] = jnp.where(live, gsel_v[sl], 0)
        sfire0[0, sl] = jnp.where(live, ssel_v[sl], junk)
        pfire0[0, sl] = jnp.where(live, psel_v[sl], 0)
    pltpu.sync_copy(x_hbm.at[gfire0.at[0]], rows0)
    pltpu.sync_copy(ea_hbm.at[pfire0.at[0]], ea0)
    relu_rows(rows0, ea0)
    pltpu.sync_copy(rows0, acc.at[sfire0.at[0]], add=True)

    pltpu.sync_copy(acc.at[pl.ds(lo, npt)], out_hbm.at[c, pl.ds(lo, npt)])


def _gelu(h):
    return 0.5 * h * (1.0 + lax.erf(h * (1.0 / math.sqrt(2.0))))


def _post_tc_kernel(x_ref, a0_ref, a1_ref, w_ref, b_ref, o_ref):
    xb = x_ref[...]
    hp = lax.Precision.HIGHEST
    hin = xb + a0_ref[0]
    hin = jnp.dot(hin, w_ref[0], preferred_element_type=jnp.float32,
                  precision=hp) + b_ref[0]
    hin = _gelu(hin)
    hin = jnp.dot(hin, w_ref[1], preferred_element_type=jnp.float32,
                  precision=hp) + b_ref[1]
    hout = xb + a1_ref[0]
    hout = jnp.dot(hout, w_ref[2], preferred_element_type=jnp.float32,
                   precision=hp) + b_ref[2]
    hout = _gelu(hout)
    hout = jnp.dot(hout, w_ref[3], preferred_element_type=jnp.float32,
                   precision=hp) + b_ref[3]
    o_ref[...] = (0.5 * (hin + hout)
                  + jnp.dot(xb, w_ref[4], preferred_element_type=jnp.float32,
                            precision=hp)
                  + b_ref[4])


def kernel(x, edge_index, edge_attr, We_in, be_in, W1_in, b1_in, W2_in, b2_in,
           We_out, be_out, W1_out, b1_out, W2_out, b2_out, Wr, br):
    n, d = x.shape
    e = edge_index.shape[1]
    de = edge_attr.shape[1]

    blk_e = 2048
    e_pad = -(-e // blk_e) * blk_e
    pad = e_pad - e
    npt = -(-n // (NS * 8)) * 8
    n_out = npt * NS

    src = edge_index[0].astype(jnp.int32)
    dst = edge_index[1].astype(jnp.int32)
    zpad = jnp.zeros((pad,), jnp.int32)
    jpad = jnp.full((pad,), n, jnp.int32)
    gidx_all = jnp.concatenate([src, zpad, dst, zpad])
    sidx_all = jnp.concatenate([dst, jpad, src, jpad])

    attr_pad = jnp.concatenate(
        [edge_attr, jnp.zeros((pad, de), jnp.float32)], axis=0)
    We_all = jnp.stack([We_in, We_out])
    be_all = jnp.stack([be_in, be_out]).reshape(2, 1, d)

    ea_all = pl.pallas_call(
        _ea_tc_kernel,
        grid=(2, e_pad // blk_e),
        in_specs=[
            pl.BlockSpec((blk_e, de), lambda dd, i: (i, 0)),
            pl.BlockSpec((1, de, d), lambda dd, i: (dd, 0, 0)),
            pl.BlockSpec((1, 1, d), lambda dd, i: (dd, 0, 0)),
        ],
        out_specs=pl.BlockSpec((1, blk_e, d), lambda dd, i: (dd, i, 0)),
        out_shape=jax.ShapeDtypeStruct((2, e_pad, d), jnp.float32),
    )(attr_pad, We_all, be_all)
    ea_flat = ea_all.reshape(2 * e_pad, d)

    sc_params = pltpu.CompilerParams()
    if "needs_layout_passes" in pltpu.CompilerParams.__dataclass_fields__:
        sc_params = dataclasses.replace(sc_params, needs_layout_passes=False)
    mesh = plsc.VectorSubcoreMesh(core_axis_name="core",
                                  subcore_axis_name="subcore")
    sc_call = pl.kernel(
        functools.partial(_sc_kernel, e_pad, npt, n_out),
        out_type=jax.ShapeDtypeStruct((2, n_out, d), jnp.float32),
        mesh=mesh,
        scratch_types=(
            [pltpu.VMEM((SBLK,), jnp.int32)] * 4
            + [pltpu.VMEM((SEL,), jnp.int32)] * 3
            + [pltpu.VMEM((1, FBLK), jnp.int32)] * 3
            + [pltpu.VMEM((FBLK, d), jnp.float32)] * 2
            + [pltpu.VMEM((1, FBLK), jnp.int32)] * 3
            + [pltpu.VMEM((FBLK, d), jnp.float32)] * 2
            + [pltpu.SMEM((4,), jnp.int32)]
            + [pltpu.SemaphoreType.DMA] * 4
            + [pltpu.VMEM_SHARED((n_out + NS, d), jnp.float32)]
        ),
        compiler_params=sc_params,
    )
    aggr = sc_call(x, gidx_all, sidx_all, ea_flat)

    W_all = jnp.stack([W1_in, W2_in, W1_out, W2_out, Wr])
    b_all = jnp.stack([b1_in, b2_in, b1_out, b2_out, br])

    blk_n = 2000
    out = pl.pallas_call(
        _post_tc_kernel,
        grid=(n // blk_n,),
        in_specs=[
            pl.BlockSpec((blk_n, d), lambda i: (i, 0)),
            pl.BlockSpec((1, blk_n, d), lambda i: (0, i, 0)),
            pl.BlockSpec((1, blk_n, d), lambda i: (1, i, 0)),
            pl.BlockSpec((5, d, d), lambda i: (0, 0, 0)),
            pl.BlockSpec((5, d), lambda i: (0, 0)),
        ],
        out_specs=pl.BlockSpec((blk_n, d), lambda i: (i, 0)),
        out_shape=jax.ShapeDtypeStruct((n, d), jnp.float32),
    )(x, aggr, aggr, W_all, b_all)
    return out


# speedup vs baseline: 1.4079x; 1.0813x over previous
import dataclasses
import functools
import math

import jax
import jax.numpy as jnp
from jax import lax
from jax.experimental import pallas as pl
from jax.experimental.pallas import tpu as pltpu
from jax.experimental.pallas import tpu_sc as plsc

NC = 2
NS = 16
L = 16
FBLK = 64
SBLK = 1024
SEL = 160


def _ea_tc_kernel(attr_ref, w_ref, b_ref, o_ref):
    o_ref[0] = (
        jnp.dot(attr_ref[...], w_ref[0], preferred_element_type=jnp.float32,
                precision=lax.Precision.HIGHEST)
        + b_ref[0, 0]
    )


def _sc_kernel(e_pad, npt, n_out,
               x_hbm, gidx_hbm, sidx_hbm, ea_hbm, out_hbm,
               gscan0, sscan0, gscan1, sscan1, gsel_v, ssel_v, psel_v,
               gfire0, sfire0, pfire0, rows0, ea0,
               gfire1, sfire1, pfire1, rows1, ea1,
               cnt_s, sem_s0, sem_s1, sem_f0, sem_f1, acc):
    c = lax.axis_index("core")
    s = lax.axis_index("subcore")
    d = x_hbm.shape[1]
    lo = s * npt
    junk = n_out + s
    lanes = jnp.arange(L, dtype=jnp.int32)
    nblk = e_pad // SBLK
    bufs = ((gfire0, sfire0, pfire0, rows0, ea0, sem_f0),
            (gfire1, sfire1, pfire1, rows1, ea1, sem_f1))

    nzb = rows0.shape[0]

    @pl.loop(0, nzb)
    def _(i):
        for j in range(d // L):
            rows0[i, pl.ds(j * L, L)] = jnp.zeros((L,), jnp.float32)

    for off in sorted({min(j * nzb, npt - nzb) for j in range(-(-npt // nzb))}):
        pltpu.sync_copy(rows0, acc.at[pl.ds(lo + off, nzb)])

    cnt_s[0] = 0
    cnt_s[1] = 0
    cnt_s[2] = 0
    cnt_s[3] = 0

    def relu_rows(rows_v, ea_v):
        @pl.loop(0, FBLK)
        def _(e):
            for j in range(d // L):
                sl = pl.ds(j * L, L)
                rows_v[e, sl] = jnp.maximum(rows_v[e, sl] + ea_v[e, sl], 0.0)

    def flush(p):
        gfire_v, sfire_v, pfire_v, rows_v, ea_v, sem = bufs[p]

        @pl.when(cnt_s[1 + p] == 1)
        def _():
            pltpu.make_async_copy(x_hbm.at[gfire_v.at[0]], rows_v, sem).wait()
            pltpu.make_async_copy(ea_hbm.at[pfire_v.at[0]], ea_v, sem).wait()
            relu_rows(rows_v, ea_v)
            pltpu.sync_copy(rows_v, acc.at[sfire_v.at[0]], add=True)
            cnt_s[1 + p] = 0

    def fire(p):
        gfire_v, sfire_v, pfire_v, rows_v, ea_v, sem = bufs[p]
        for j in range(FBLK // L):
            sl = pl.ds(j * L, L)
            gfire_v[0, sl] = gsel_v[sl]
            sfire_v[0, sl] = ssel_v[sl]
            pfire_v[0, sl] = psel_v[sl]
        pltpu.async_copy(x_hbm.at[gfire_v.at[0]], rows_v, sem)
        pltpu.async_copy(ea_hbm.at[pfire_v.at[0]], ea_v, sem)
        for j in range((SEL - FBLK) // L):
            sl = pl.ds(j * L, L)
            gsel_v[sl] = gsel_v[pl.ds(FBLK + j * L, L)]
            ssel_v[sl] = ssel_v[pl.ds(FBLK + j * L, L)]
            psel_v[sl] = psel_v[pl.ds(FBLK + j * L, L)]
        cnt_s[0] = cnt_s[0] - FBLK
        cnt_s[1 + p] = 1

    def maybe_fire():
        @pl.when(cnt_s[0] >= FBLK)
        def _():
            par = cnt_s[3]

            @pl.when(par == 0)
            def _():
                flush(0)
                fire(0)
                cnt_s[3] = 1

            @pl.when(par == 1)
            def _():
                flush(1)
                fire(1)
                cnt_s[3] = 0

    def scan_block(blk, gscan_v, sscan_v):
        @pl.loop(0, SBLK // 128)
        def _(sb):
            for half in range(2):
                work = []
                for k in range(4):
                    off = sb * 128 + half * 64 + k * L
                    dstv = sscan_v[pl.ds(off, L)]
                    srcv = gscan_v[pl.ds(off, L)]
                    m = (dstv >= lo) & (dstv < lo + npt)
                    pc = plsc.all_reduce_population_count(m)[0]
                    posv = (c * e_pad + blk * SBLK + off) + lanes
                    work.append((m, srcv, dstv, posv, pc))
                ptr = cnt_s[0]
                for m, srcv, dstv, posv, pc in work:
                    plsc.store_compressed(gsel_v.at[pl.ds(ptr, L)], srcv,
                                          mask=m)
                    plsc.store_compressed(ssel_v.at[pl.ds(ptr, L)], dstv,
                                          mask=m)
                    plsc.store_compressed(psel_v.at[pl.ds(ptr, L)], posv,
                                          mask=m)
                    ptr = ptr + pc
                cnt_s[0] = ptr
                maybe_fire()

    def scan_issue(blk, gscan_v, sscan_v, sem):
        pltpu.async_copy(
            gidx_hbm.at[pl.ds(c * e_pad + blk * SBLK, SBLK)], gscan_v, sem)
        pltpu.async_copy(
            sidx_hbm.at[pl.ds(c * e_pad + blk * SBLK, SBLK)], sscan_v, sem)

    def scan_wait(blk, gscan_v, sscan_v, sem):
        pltpu.make_async_copy(
            gidx_hbm.at[pl.ds(c * e_pad + blk * SBLK, SBLK)], gscan_v,
            sem).wait()
        pltpu.make_async_copy(
            sidx_hbm.at[pl.ds(c * e_pad + blk * SBLK, SBLK)], sscan_v,
            sem).wait()

    scan_issue(0, gscan0, sscan0, sem_s0)

    @pl.loop(0, nblk // 2)
    def _(i):
        blk0 = 2 * i
        scan_wait(blk0, gscan0, sscan0, sem_s0)
        scan_issue(blk0 + 1, gscan1, sscan1, sem_s1)
        scan_block(blk0, gscan0, sscan0)
        scan_wait(blk0 + 1, gscan1, sscan1, sem_s1)

        @pl.when(blk0 + 2 < nblk)
        def _():
            scan_issue(blk0 + 2, gscan0, sscan0, sem_s0)

        scan_block(blk0 + 1, gscan1, sscan1)

    flush(0)
    flush(1)
    cnt = cnt_s[0]
    for j in range(FBLK // L):
        sl = pl.ds(j * L, L)
        live = (lanes + (j * L)) < cnt
        gfire0[0, sl] = jnp.where(live, gsel_v[sl], 0)
        sfire0[0, sl] = jnp.where(live, ssel_v[sl], junk)
        pfire0[0, sl] = jnp.where(live, psel_v[sl], 0)
    pltpu.sync_copy(x_hbm.at[gfire0.at[0]], rows0)
    pltpu.sync_copy(ea_hbm.at[pfire0.at[0]], ea0)
    relu_rows(rows0, ea0)
    pltpu.sync_copy(rows0, acc.at[sfire0.at[0]], add=True)

    pltpu.sync_copy(acc.at[pl.ds(lo, npt)], out_hbm.at[c, pl.ds(lo, npt)])


def _gelu(h):
    return 0.5 * h * (1.0 + lax.erf(h * (1.0 / math.sqrt(2.0))))


def _post_tc_kernel(x_ref, a0_ref, a1_ref, w_ref, b_ref, o_ref):
    xb = x_ref[...]
    hp = lax.Precision.HIGHEST
    hin = xb + a0_ref[0]
    hin = jnp.dot(hin, w_ref[0], preferred_element_type=jnp.float32,
                  precision=hp) + b_ref[0]
    hin = _gelu(hin)
    hin = jnp.dot(hin, w_ref[1], preferred_element_type=jnp.float32,
                  precision=hp) + b_ref[1]
    hout = xb + a1_ref[0]
    hout = jnp.dot(hout, w_ref[2], preferred_element_type=jnp.float32,
                   precision=hp) + b_ref[2]
    hout = _gelu(hout)
    hout = jnp.dot(hout, w_ref[3], preferred_element_type=jnp.float32,
                   precision=hp) + b_ref[3]
    o_ref[...] = (0.5 * (hin + hout)
                  + jnp.dot(xb, w_ref[4], preferred_element_type=jnp.float32,
                            precision=hp)
                  + b_ref[4])


def kernel(x, edge_index, edge_attr, We_in, be_in, W1_in, b1_in, W2_in, b2_in,
           We_out, be_out, W1_out, b1_out, W2_out, b2_out, Wr, br):
    n, d = x.shape
    e = edge_index.shape[1]
    de = edge_attr.shape[1]

    blk_e = 2048
    e_pad = -(-e // blk_e) * blk_e
    pad = e_pad - e
    npt = -(-n // (NS * 8)) * 8
    n_out = npt * NS

    src = edge_index[0].astype(jnp.int32)
    dst = edge_index[1].astype(jnp.int32)
    zpad = jnp.zeros((pad,), jnp.int32)
    jpad = jnp.full((pad,), n, jnp.int32)
    gidx_all = jnp.concatenate([src, zpad, dst, zpad])
    sidx_all = jnp.concatenate([dst, jpad, src, jpad])

    attr_pad = jnp.concatenate(
        [edge_attr, jnp.zeros((pad, de), jnp.float32)], axis=0)
    We_all = jnp.stack([We_in, We_out])
    be_all = jnp.stack([be_in, be_out]).reshape(2, 1, d)

    ea_all = pl.pallas_call(
        _ea_tc_kernel,
        grid=(2, e_pad // blk_e),
        in_specs=[
            pl.BlockSpec((blk_e, de), lambda dd, i: (i, 0)),
            pl.BlockSpec((1, de, d), lambda dd, i: (dd, 0, 0)),
            pl.BlockSpec((1, 1, d), lambda dd, i: (dd, 0, 0)),
        ],
        out_specs=pl.BlockSpec((1, blk_e, d), lambda dd, i: (dd, i, 0)),
        out_shape=jax.ShapeDtypeStruct((2, e_pad, d), jnp.float32),
    )(attr_pad, We_all, be_all)
    ea_flat = ea_all.reshape(2 * e_pad, d)

    sc_params = pltpu.CompilerParams()
    if "needs_layout_passes" in pltpu.CompilerParams.__dataclass_fields__:
        sc_params = dataclasses.replace(sc_params, needs_layout_passes=False)
    mesh = plsc.VectorSubcoreMesh(core_axis_name="core",
                                  subcore_axis_name="subcore")
    sc_call = pl.kernel(
        functools.partial(_sc_kernel, e_pad, npt, n_out),
        out_type=jax.ShapeDtypeStruct((2, n_out, d), jnp.float32),
        mesh=mesh,
        scratch_types=(
            [pltpu.VMEM((SBLK,), jnp.int32)] * 4
            + [pltpu.VMEM((SEL,), jnp.int32)] * 3
            + [pltpu.VMEM((1, FBLK), jnp.int32)] * 3
            + [pltpu.VMEM((FBLK, d), jnp.float32)] * 2
            + [pltpu.VMEM((1, FBLK), jnp.int32)] * 3
            + [pltpu.VMEM((FBLK, d), jnp.float32)] * 2
            + [pltpu.SMEM((4,), jnp.int32)]
            + [pltpu.SemaphoreType.DMA] * 4
            + [pltpu.VMEM_SHARED((n_out + NS, d), jnp.float32)]
        ),
        compiler_params=sc_params,
    )
    aggr = sc_call(x, gidx_all, sidx_all, ea_flat)

    W_all = jnp.stack([W1_in, W2_in, W1_out, W2_out, Wr])
    b_all = jnp.stack([b1_in, b2_in, b1_out, b2_out, br])

    blk_n = 2000
    out = pl.pallas_call(
        _post_tc_kernel,
        grid=(n // blk_n,),
        in_specs=[
            pl.BlockSpec((blk_n, d), lambda i: (i, 0)),
            pl.BlockSpec((1, blk_n, d), lambda i: (0, i, 0)),
            pl.BlockSpec((1, blk_n, d), lambda i: (1, i, 0)),
            pl.BlockSpec((5, d, d), lambda i: (0, 0, 0)),
            pl.BlockSpec((5, d), lambda i: (0, 0)),
        ],
        out_specs=pl.BlockSpec((blk_n, d), lambda i: (i, 0)),
        out_shape=jax.ShapeDtypeStruct((n, d), jnp.float32),
    )(x, aggr, aggr, W_all, b_all)
    return out
